# baseline (speedup 1.0000x reference)
import functools

import jax
import jax.numpy as jnp
from jax import lax
from jax.experimental import pallas as pl
from jax.experimental.pallas import tpu as pltpu
from jax.experimental.pallas import tpu_sc as plsc

NC = 2
NS = 16
NW = NC * NS
CHUNK = 128


def _make_sc_gather(V, D, B):
    bpw = B // NW
    nch = bpw // CHUNK
    mesh = plsc.VectorSubcoreMesh(core_axis_name="c", subcore_axis_name="s")

    @functools.partial(
        pl.kernel,
        mesh=mesh,
        compiler_params=pltpu.CompilerParams(use_tc_tiling_on_sc=False),
        out_type=jax.ShapeDtypeStruct((NW, bpw, D), jnp.float32),
        scratch_types=[
            pltpu.VMEM((nch, CHUNK), jnp.int32),
            pltpu.VMEM((bpw, D), jnp.float32),
            pltpu.SemaphoreType.DMA,
        ],
    )
    def gather_k(y_hbm, emb_hbm, g_hbm, idx_v, rows_v, sem):
        wid = lax.axis_index("s") * NC + lax.axis_index("c")
        pltpu.sync_copy(y_hbm.at[wid], idx_v)
        copies = []
        for j in range(nch):
            copies.append(
                pltpu.async_copy(
                    emb_hbm.at[idx_v.at[j]],
                    rows_v.at[pl.ds(j * CHUNK, CHUNK)],
                    sem,
                )
            )
        for cp in copies:
            cp.wait()
        pltpu.sync_copy(rows_v, g_hbm.at[wid])

    return gather_k


def _tc_combine(zT, g):
    D, B = zT.shape
    blk = 2048

    def body(z_ref, g_ref, o_ref):
        zb = z_ref[...]
        s = jnp.sum(zb * zb, axis=0, keepdims=True)
        eye = jnp.eye(D, dtype=jnp.float32)
        gt = lax.dot_general(
            eye, g_ref[...], (((1,), (1,)), ((), ())),
            precision=lax.Precision.HIGHEST,
        )
        o_ref[...] = zb * lax.rsqrt(s) + gt

    return pl.pallas_call(
        body,
        grid=(B // blk,),
        in_specs=[
            pl.BlockSpec((D, blk), lambda i: (0, i)),
            pl.BlockSpec((blk, D), lambda i: (i, 0)),
        ],
        out_specs=pl.BlockSpec((D, blk), lambda i: (0, i)),
        out_shape=jax.ShapeDtypeStruct((D, B), jnp.float32),
    )(zT, g)


def kernel(z, y, embedding):
    B, D = z.shape
    V = embedding.shape[0]
    bpw = B // NW
    y3 = y.astype(jnp.int32).reshape(NW, bpw // CHUNK, CHUNK)
    g = _make_sc_gather(V, D, B)(y3, embedding).reshape(B, D)
    outT = _tc_combine(z.T, g)
    return outT.T


# speedup vs baseline: 1.8297x; 1.8297x over previous
import functools

import jax
import jax.numpy as jnp
from jax import lax
from jax.experimental import pallas as pl
from jax.experimental.pallas import tpu as pltpu
from jax.experimental.pallas import tpu_sc as plsc

NC = 2
NS = 16
NW = NC * NS
L = 16
PW = 128


def _make_sc_gather(V, D, B):
    bpw = B // NW

    mesh = plsc.VectorSubcoreMesh(core_axis_name="c", subcore_axis_name="s")

    @functools.partial(
        pl.kernel,
        mesh=mesh,
        compiler_params=pltpu.CompilerParams(needs_layout_passes=False),
        out_type=jax.ShapeDtypeStruct((B, PW), jnp.float32),
        scratch_types=[
            pltpu.VMEM((bpw // PW, PW), jnp.int32),
            pltpu.VMEM((2, D, PW), jnp.float32),
            pltpu.VMEM((bpw, PW), jnp.float32),
            pltpu.SemaphoreType.DMA,
            pltpu.SemaphoreType.DMA,
        ],
    )
    def gather_k(y_hbm, embT_hbm, g_hbm, idx_v, panels_v, rows_v, sem0, sem1):
        wid = lax.axis_index("s") * NC + lax.axis_index("c")
        base = wid * bpw
        pltpu.sync_copy(y_hbm.at[wid], idx_v)

        lanes = lax.iota(jnp.int32, L)

        def scalar_idx(r):
            g = lax.shift_right_logical(r, 4)
            vec = idx_v[lax.shift_right_logical(g, 3),
                        pl.ds(pl.multiple_of((g & 7) * L, L), L)]
            j = jnp.full((L,), r & (L - 1), jnp.int32)
            return vec.at[j].get(mode="promise_in_bounds")[0]

        def fire(r, buf, sem):
            i = scalar_idx(r)
            start = pl.multiple_of(i & ~jnp.int32(PW - 1), PW)
            pltpu.async_copy(
                embT_hbm.at[:, pl.ds(start, PW)], panels_v.at[buf], sem
            )

        def drain(buf, sem):
            pltpu.make_async_copy(
                embT_hbm.at[:, pl.ds(0, PW)], panels_v.at[buf], sem
            ).wait()

        def extract(r, buf):
            i = scalar_idx(r)
            col = jnp.full((L,), i & (PW - 1), jnp.int32)
            for k in range(D // L):
                row_idx = lanes + (L * k)
                q = plsc.load_gather(panels_v.at[buf], [row_idx, col])
                rows_v[r, pl.ds(L * k, L)] = q

        fire(jnp.int32(0), 0, sem0)

        def pair_body(rp, carry):
            r0 = rp * 2
            fire(r0 + 1, 1, sem1)
            drain(0, sem0)
            extract(r0, 0)

            @pl.when(r0 + 2 < bpw)
            def _():
                fire(r0 + 2, 0, sem0)

            drain(1, sem1)
            extract(r0 + 1, 1)
            return carry

        lax.fori_loop(0, bpw // 2, pair_body, 0)
        pltpu.sync_copy(rows_v, g_hbm.at[pl.ds(base, bpw)])

    return gather_k


def _tc_combine(zT, g):
    D, B = zT.shape
    blk = 2048

    def body(z_ref, g_ref, o_ref):
        zb = z_ref[...]
        s = jnp.sum(zb * zb, axis=0, keepdims=True)
        eye = jnp.eye(D, dtype=jnp.float32)
        gt = lax.dot_general(
            eye, g_ref[..., :D], (((1,), (1,)), ((), ())),
            precision=lax.Precision.HIGHEST,
        )
        o_ref[...] = zb * lax.rsqrt(s) + gt

    return pl.pallas_call(
        body,
        grid=(B // blk,),
        in_specs=[
            pl.BlockSpec((D, blk), lambda i: (0, i)),
            pl.BlockSpec((blk, PW), lambda i: (i, 0)),
        ],
        out_specs=pl.BlockSpec((D, blk), lambda i: (0, i)),
        out_shape=jax.ShapeDtypeStruct((D, B), jnp.float32),
    )(zT, g)


def kernel(z, y, embedding):
    B, D = z.shape
    V = embedding.shape[0]
    bpw = B // NW
    y3 = y.astype(jnp.int32).reshape(NW, bpw // PW, PW)
    g = _make_sc_gather(V, D, B)(y3, embedding.T)
    outT = _tc_combine(z.T, g)
    return outT.T


# speedup vs baseline: 2.4784x; 1.3545x over previous
import functools

import jax
import jax.numpy as jnp
from jax import lax
from jax.experimental import pallas as pl
from jax.experimental.pallas import tpu as pltpu
from jax.experimental.pallas import tpu_sc as plsc

NC = 2
NS = 16
NW = NC * NS
L = 16
PW = 128
NBUF = 4


def _make_sc_gather(V, D, B):
    bpw = B // NW

    mesh = plsc.VectorSubcoreMesh(core_axis_name="c", subcore_axis_name="s")

    @functools.partial(
        pl.kernel,
        mesh=mesh,
        compiler_params=pltpu.CompilerParams(needs_layout_passes=False),
        out_type=jax.ShapeDtypeStruct((B, PW), jnp.float32),
        scratch_types=[
            pltpu.VMEM((bpw // PW, PW), jnp.int32),
            pltpu.VMEM((NBUF, D, PW), jnp.float32),
            pltpu.VMEM((bpw, PW), jnp.float32),
            [pltpu.SemaphoreType.DMA] * NBUF,
        ],
    )
    def gather_k(y_hbm, embT_hbm, g_hbm, idx_v, panels_v, rows_v, sems):
        wid = lax.axis_index("s") * NC + lax.axis_index("c")
        base = wid * bpw
        pltpu.sync_copy(y_hbm.at[wid], idx_v)

        lanes = lax.iota(jnp.int32, L)

        def scalar_idx(r):
            g = lax.shift_right_logical(r, 4)
            vec = idx_v[lax.shift_right_logical(g, 3),
                        pl.ds(pl.multiple_of((g & 7) * L, L), L)]
            j = jnp.full((L,), r & (L - 1), jnp.int32)
            return vec.at[j].get(mode="promise_in_bounds")[0]

        def fire(r, buf, sem):
            i = scalar_idx(r)
            start = pl.multiple_of(i & ~jnp.int32(PW - 1), PW)
            pltpu.async_copy(
                embT_hbm.at[:, pl.ds(start, PW)], panels_v.at[buf], sem
            )

        def drain(buf, sem):
            pltpu.make_async_copy(
                embT_hbm.at[:, pl.ds(0, PW)], panels_v.at[buf], sem
            ).wait()

        def extract(r, buf):
            i = scalar_idx(r)
            col = jnp.full((L,), i & (PW - 1), jnp.int32)
            for k in range(D // L):
                row_idx = lanes + (L * k)
                q = plsc.load_gather(panels_v.at[buf], [row_idx, col])
                rows_v[r, pl.ds(L * k, L)] = q

        for p in range(NBUF - 1):
            fire(jnp.int32(p), p, sems[p])

        def quad_body(rq, carry):
            r0 = rq * NBUF
            for p in range(NBUF):
                r = r0 + p
                nb = (p + NBUF - 1) % NBUF

                @pl.when(r + NBUF - 1 < bpw)
                def _():
                    fire(r + NBUF - 1, nb, sems[nb])

                drain(p, sems[p])
                extract(r, p)
            return carry

        lax.fori_loop(0, bpw // NBUF, quad_body, 0)
        pltpu.sync_copy(rows_v, g_hbm.at[pl.ds(base, bpw)])

    return gather_k


def _tc_combine(zT, g):
    D, B = zT.shape
    blk = 2048

    def body(z_ref, g_ref, o_ref):
        zb = z_ref[...]
        s = jnp.sum(zb * zb, axis=0, keepdims=True)
        eye = jnp.eye(D, dtype=jnp.float32)
        gt = lax.dot_general(
            eye, g_ref[..., :D], (((1,), (1,)), ((), ())),
            precision=lax.Precision.HIGHEST,
        )
        o_ref[...] = zb * lax.rsqrt(s) + gt

    return pl.pallas_call(
        body,
        grid=(B // blk,),
        in_specs=[
            pl.BlockSpec((D, blk), lambda i: (0, i)),
            pl.BlockSpec((blk, PW), lambda i: (i, 0)),
        ],
        out_specs=pl.BlockSpec((D, blk), lambda i: (0, i)),
        out_shape=jax.ShapeDtypeStruct((D, B), jnp.float32),
    )(zT, g)


def kernel(z, y, embedding):
    B, D = z.shape
    V = embedding.shape[0]
    bpw = B // NW
    y3 = y.astype(jnp.int32).reshape(NW, bpw // PW, PW)
    g = _make_sc_gather(V, D, B)(y3, embedding.T)
    outT = _tc_combine(z.T, g)
    return outT.T


# speedup vs baseline: 2.8835x; 1.1634x over previous
import functools

import jax
import jax.numpy as jnp
from jax import lax
from jax.experimental import pallas as pl
from jax.experimental.pallas import tpu as pltpu
from jax.experimental.pallas import tpu_sc as plsc

NC = 2
NS = 16
NW = NC * NS
L = 16
PW = 128
NBUF = 8
HALF = 256


def _make_sc_gather(V, D, B):
    bpw = B // NW

    mesh = plsc.VectorSubcoreMesh(core_axis_name="c", subcore_axis_name="s")

    @functools.partial(
        pl.kernel,
        mesh=mesh,
        compiler_params=pltpu.CompilerParams(needs_layout_passes=False),
        out_type=jax.ShapeDtypeStruct((B, PW), jnp.float32),
        scratch_types=[
            pltpu.VMEM((bpw // PW, PW), jnp.int32),
            pltpu.VMEM((NBUF, D, PW), jnp.float32),
            pltpu.VMEM((HALF, PW), jnp.float32),
            [pltpu.SemaphoreType.DMA] * NBUF,
        ],
    )
    def gather_k(y_hbm, embT_hbm, g_hbm, idx_v, panels_v, rows_v, sems):
        wid = lax.axis_index("s") * NC + lax.axis_index("c")
        base = wid * bpw
        pltpu.sync_copy(y_hbm.at[wid], idx_v)

        lanes = lax.iota(jnp.int32, L)

        def scalar_idx(r):
            g = lax.shift_right_logical(r, 4)
            vec = idx_v[lax.shift_right_logical(g, 3),
                        pl.ds(pl.multiple_of((g & 7) * L, L), L)]
            j = jnp.full((L,), r & (L - 1), jnp.int32)
            return vec.at[j].get(mode="promise_in_bounds")[0]

        def fire(r, buf, sem):
            i = scalar_idx(r)
            start = pl.multiple_of(i & ~jnp.int32(PW - 1), PW)
            pltpu.async_copy(
                embT_hbm.at[:, pl.ds(start, PW)], panels_v.at[buf], sem
            )

        def drain(buf, sem):
            pltpu.make_async_copy(
                embT_hbm.at[:, pl.ds(0, PW)], panels_v.at[buf], sem
            ).wait()

        def extract(r, buf):
            i = scalar_idx(r)
            col = jnp.full((L,), i & (PW - 1), jnp.int32)
            for k in range(D // L):
                row_idx = lanes + (L * k)
                q = plsc.load_gather(panels_v.at[buf], [row_idx, col])
                rows_v[r & (HALF - 1), pl.ds(L * k, L)] = q

        for h in range(bpw // HALF):
            r_lo = h * HALF
            r_hi = r_lo + HALF
            for p in range(NBUF - 1):
                fire(jnp.int32(r_lo + p), p, sems[p])

            def quad_body(rq, carry):
                r0 = r_lo + rq * NBUF
                for p in range(NBUF):
                    r = r0 + p
                    nb = (p + NBUF - 1) % NBUF

                    @pl.when(r + NBUF - 1 < r_hi)
                    def _():
                        fire(r + NBUF - 1, nb, sems[nb])

                    drain(p, sems[p])
                    extract(r, p)
                return carry

            lax.fori_loop(0, HALF // NBUF, quad_body, 0)
            pltpu.sync_copy(rows_v, g_hbm.at[pl.ds(base + r_lo, HALF)])

    return gather_k


def _tc_combine(zT, g):
    D, B = zT.shape
    blk = 2048

    def body(z_ref, g_ref, o_ref):
        zb = z_ref[...]
        s = jnp.sum(zb * zb, axis=0, keepdims=True)
        eye = jnp.eye(D, dtype=jnp.float32)
        gt = lax.dot_general(
            eye, g_ref[..., :D], (((1,), (1,)), ((), ())),
            precision=lax.Precision.HIGHEST,
        )
        o_ref[...] = zb * lax.rsqrt(s) + gt

    return pl.pallas_call(
        body,
        grid=(B // blk,),
        in_specs=[
            pl.BlockSpec((D, blk), lambda i: (0, i)),
            pl.BlockSpec((blk, PW), lambda i: (i, 0)),
        ],
        out_specs=pl.BlockSpec((D, blk), lambda i: (0, i)),
        out_shape=jax.ShapeDtypeStruct((D, B), jnp.float32),
    )(zT, g)


def kernel(z, y, embedding):
    B, D = z.shape
    V = embedding.shape[0]
    bpw = B // NW
    y3 = y.astype(jnp.int32).reshape(NW, bpw // PW, PW)
    g = _make_sc_gather(V, D, B)(y3, embedding.T)
    outT = _tc_combine(z.T, g)
    return outT.T
